# baseline (speedup 1.0000x reference)
import functools

import jax
import jax.numpy as jnp
from jax.experimental import pallas as pl
from jax.experimental.pallas import tpu as pltpu

B, L, P, T = 16, 32, 32, 512
H, R, OUT = 1024, 1024, 1024
NBUF = 4

_DN = (((1,), (1,)), ((), ()))


def _dott(x, w):
    return jax.lax.dot_general(x, w, _DN, preferred_element_type=jnp.float32)


def _dotb(x, w):
    return jax.lax.dot_general(x.astype(jnp.bfloat16), w, _DN,
                               preferred_element_type=jnp.float32)


def _iota(shape, dim):
    return jax.lax.broadcasted_iota(jnp.int32, shape, dim)


def _row(ref):
    return ref[...].reshape(1, -1)


def _mega_kernel(A_ref, V_ref, od_ref, id_ref,
                 gl_ref, lp_ref, olen_ref, ilen_ref,
                 twA_ref, tbA_ref, twV_ref, tbV_ref,
                 twO_ref, tbO_ref, twI_ref, tbI_ref,
                 b1A_ref, b2A_ref, b3A_ref, b1V_ref, b2V_ref, b3V_ref,
                 b1O_ref, b2O_ref, b3O_ref, b1I_ref, b2I_ref, b3I_ref,
                 bih_ref, bsi_ref, bhh_ref, bo_ref,
                 w1a, w1v, w1o, w1i, w2a, w3a, w2v, w3v,
                 w2o, w3o, w2i, w3i, wih, wo, wsi, whh,
                 out_ref, wbuf, wih_buf, wo_buf, xs, sems, sem_ih, sem_o):
    srcs = [w1a, w1v, w1i,
            w2a, w2v, w3a, w3v,
            w2i, w3i, wsi,
            whh,
            w1o, w2o, w3o]
    n_uses = len(srcs)

    cp_ih = pltpu.make_async_copy(wih, wih_buf, sem_ih)
    cp_o = pltpu.make_async_copy(wo, wo_buf, sem_o)

    def start(k):
        if k < n_uses:
            pltpu.make_async_copy(srcs[k], wbuf.at[k % NBUF], sems.at[k % NBUF]).start()

    def wait(k):
        pltpu.make_async_copy(srcs[k], wbuf.at[k % NBUF], sems.at[k % NBUF]).wait()
        return wbuf[k % NBUF]

    for k in range(NBUF):
        start(k)

    glr = gl_ref[...].reshape(1, B)
    lpr = lp_ref[...].reshape(1, B)
    glc = gl_ref[...].reshape(B, 1)
    lmask = (_iota((L, B), 0) < glr).astype(jnp.float32)
    c_lb = lmask * lpr.astype(jnp.float32)
    pmaskT = (_iota((B, L, P), 2)
              < lp_ref[...].reshape(B, 1, 1)).astype(jnp.float32)
    sA = jnp.sum(A_ref[...] * pmaskT, axis=2).T * lmask
    sV = jnp.sum(V_ref[...] * pmaskT, axis=2).T * lmask

    olen = olen_ref[...].reshape(B, 1)
    ilen = ilen_ref[...].reshape(B, 1)
    omask = (_iota((B, T), 1) < olen).astype(jnp.float32)
    imask = (_iota((B, T), 1) < ilen).astype(jnp.float32)
    sO = jnp.sum(od_ref[...] * omask, axis=1, keepdims=True)
    sI = jnp.sum(id_ref[...] * imask, axis=1, keepdims=True)
    cO = olen.astype(jnp.float32)
    cI = ilen.astype(jnp.float32)

    def twtb(tw_ref, tb_ref):
        return jnp.concatenate([_row(tw_ref), _row(tb_ref)], axis=0)

    def h1_layer(s_lb, uv, b1_ref):
        h = (s_lb[:, :, None] * uv[0:1, :][None]
             + c_lb[:, :, None] * uv[1:2, :][None] + _row(b1_ref)[None])
        return jnp.maximum(h, 0.0).reshape(L * B, H)

    w = wait(0); uvA = _dott(twtb(twA_ref, tbA_ref), w); start(4)
    w = wait(1); uvV = _dott(twtb(twV_ref, tbV_ref), w); start(5)
    w = wait(2); uvI = _dott(twtb(twI_ref, tbI_ref), w); start(6)

    h1A = h1_layer(sA, uvA, b1A_ref)
    h1V = h1_layer(sV, uvV, b1V_ref)
    w = wait(3); h2A = jnp.maximum(_dotb(h1A, w) + _row(b2A_ref), 0.0); cp_ih.start()
    w = wait(4); h2V = jnp.maximum(_dotb(h1V, w) + _row(b2V_ref), 0.0); start(7)
    w = wait(5); encA = _dotb(h2A, w) + _row(b3A_ref); start(8)
    w = wait(6); encV = _dotb(h2V, w) + _row(b3V_ref); start(9)
    cp_ih.wait()
    xs[...] = (_dotb(encA, wih_buf[:, :H]) + _dotb(encV, wih_buf[:, H:])
               + _row(bih_ref))

    h1e = jnp.maximum(sI * uvI[0:1, :] + cI * uvI[1:2, :] + _row(b1I_ref), 0.0)
    w = wait(7); h2e = jnp.maximum(_dotb(h1e, w) + _row(b2I_ref), 0.0); start(10)
    w = wait(8); embI = _dotb(h2e, w) + _row(b3I_ref); start(11)
    w = wait(9); state = _dotb(embI, w) + _row(bsi_ref); start(12)

    whh_v = wait(10); start(13)
    cp_o.start()
    bhh = _row(bhh_ref)

    def step(t, h):
        x_t = xs[t * B:(t + 1) * B, :]
        hn = jnp.tanh(x_t + _dott(h, whh_v) + bhh)
        return jnp.where(glc > t, hn, h)

    h = state
    for t in range(0, 10):
        h = step(t, h)
    wn = wait(11); uvO = _dott(twtb(twO_ref, tbO_ref), wn)
    h1eO = jnp.maximum(sO * uvO[0:1, :] + cO * uvO[1:2, :] + _row(b1O_ref), 0.0)
    for t in range(10, 18):
        h = step(t, h)
    wn = wait(12); h2eO = jnp.maximum(_dotb(h1eO, wn) + _row(b2O_ref), 0.0)
    for t in range(18, 26):
        h = step(t, h)
    wn = wait(13); embO = _dotb(h2eO, wn) + _row(b3O_ref)
    for t in range(26, L):
        h = step(t, h)
    cp_o.wait()
    outpart = _dotb(embO, wo_buf[:, R:])

    out_ref[...] = _dott(h, wo_buf[:, :R]) + outpart + _row(bo_ref)


@functools.partial(jax.jit, static_argnames=())
def kernel(A_data, V_data, output_data, input_data, gnn_layers, layer_parameters,
           output_lengths, input_lengths, params):
    p = params
    f32 = jnp.float32

    vspec = pl.BlockSpec(memory_space=pltpu.MemorySpace.VMEM)
    hspec = pl.BlockSpec(memory_space=pltpu.MemorySpace.HBM)

    small_ops = (
        A_data, V_data, output_data, input_data,
        gnn_layers.astype(jnp.int32), layer_parameters.astype(jnp.int32),
        output_lengths.astype(jnp.int32), input_lengths.astype(jnp.int32),
        p['A_tw'], p['A_tb'], p['V_tw'], p['V_tb'],
        p['out_tw'], p['out_tb'], p['inp_tw'], p['inp_tb'],
        p['A_b1'], p['A_b2'], p['A_b3'],
        p['V_b1'], p['V_b2'], p['V_b3'],
        p['out_b1'], p['out_b2'], p['out_b3'],
        p['inp_b1'], p['inp_b2'], p['inp_b3'],
        p['bih'], p['bsi'], p['bhh'], p['bo'],
    )
    big_ops = (
        p['A_W1'], p['V_W1'], p['out_W1'], p['inp_W1'],
        p['A_W2'], p['A_W3'], p['V_W2'], p['V_W3'],
        p['out_W2'], p['out_W3'], p['inp_W2'], p['inp_W3'],
        p['Wih'], p['Wo'], p['Wsi'], p['Whh'],
    )

    return pl.pallas_call(
        _mega_kernel,
        out_shape=jax.ShapeDtypeStruct((B, OUT), f32),
        in_specs=[vspec] * len(small_ops) + [hspec] * len(big_ops),
        out_specs=vspec,
        scratch_shapes=[
            pltpu.VMEM((NBUF, H, H), f32),
            pltpu.VMEM((R, 2 * H), f32),
            pltpu.VMEM((OUT, 2 * H), f32),
            pltpu.VMEM((L * B, R), f32),
            pltpu.SemaphoreType.DMA((NBUF,)),
            pltpu.SemaphoreType.DMA,
            pltpu.SemaphoreType.DMA,
        ],
        compiler_params=pltpu.CompilerParams(
            vmem_limit_bytes=100 * 1024 * 1024,
        ),
    )(*small_ops, *big_ops)


# speedup vs baseline: 1.0128x; 1.0128x over previous
import functools

import jax
import jax.numpy as jnp
from jax.experimental import pallas as pl
from jax.experimental.pallas import tpu as pltpu

B, L, P, T = 16, 32, 32, 512
H, R, OUT = 1024, 1024, 1024
NBUF = 4

_DN = (((1,), (1,)), ((), ()))


def _dott(x, w):
    return jax.lax.dot_general(x, w, _DN, preferred_element_type=jnp.float32)


def _iota(shape, dim):
    return jax.lax.broadcasted_iota(jnp.int32, shape, dim)


def _row(ref):
    return ref[...].reshape(1, -1)


def _mega_kernel(A_ref, V_ref, od_ref, id_ref,
                 gl_ref, lp_ref, olen_ref, ilen_ref,
                 twA_ref, tbA_ref, twV_ref, tbV_ref,
                 twO_ref, tbO_ref, twI_ref, tbI_ref,
                 b1A_ref, b2A_ref, b3A_ref, b1V_ref, b2V_ref, b3V_ref,
                 b1O_ref, b2O_ref, b3O_ref, b1I_ref, b2I_ref, b3I_ref,
                 bih_ref, bsi_ref, bhh_ref, bo_ref,
                 w1a, w1v, w1o, w1i, w2a, w3a, w2v, w3v,
                 w2o, w3o, w2i, w3i, wih, wo, wsi, whh,
                 out_ref, wbuf, wih_buf, wo_buf, xs, sems, sem_ih, sem_o):
    srcs = [w1a, w1v, w1i,
            w2a, w2v, w3a, w3v,
            w2i, w3i, wsi,
            whh,
            w1o, w2o, w3o]
    n_uses = len(srcs)

    cp_ih = pltpu.make_async_copy(wih, wih_buf, sem_ih)
    cp_o = pltpu.make_async_copy(wo, wo_buf, sem_o)

    def start(k):
        if k < n_uses:
            pltpu.make_async_copy(srcs[k], wbuf.at[k % NBUF], sems.at[k % NBUF]).start()

    def wait(k):
        pltpu.make_async_copy(srcs[k], wbuf.at[k % NBUF], sems.at[k % NBUF]).wait()
        return wbuf[k % NBUF]

    for k in range(NBUF):
        start(k)

    glr = gl_ref[...].reshape(1, B)
    lpr = lp_ref[...].reshape(1, B)
    glc = gl_ref[...].reshape(B, 1)
    lmask = (_iota((L, B), 0) < glr).astype(jnp.float32)
    c_lb = lmask * lpr.astype(jnp.float32)
    pmaskT = (_iota((B, L, P), 2)
              < lp_ref[...].reshape(B, 1, 1)).astype(jnp.float32)
    sA = jnp.sum(A_ref[...] * pmaskT, axis=2).T * lmask
    sV = jnp.sum(V_ref[...] * pmaskT, axis=2).T * lmask

    olen = olen_ref[...].reshape(B, 1)
    ilen = ilen_ref[...].reshape(B, 1)
    omask = (_iota((B, T), 1) < olen).astype(jnp.float32)
    imask = (_iota((B, T), 1) < ilen).astype(jnp.float32)
    sO = jnp.sum(od_ref[...] * omask, axis=1, keepdims=True)
    sI = jnp.sum(id_ref[...] * imask, axis=1, keepdims=True)
    cO = olen.astype(jnp.float32)
    cI = ilen.astype(jnp.float32)

    def twtb(tw_ref, tb_ref):
        return jnp.concatenate([_row(tw_ref), _row(tb_ref)], axis=0)

    def h1_layer(s_lb, uv, b1_ref):
        h = (s_lb[:, :, None] * uv[0:1, :][None]
             + c_lb[:, :, None] * uv[1:2, :][None] + _row(b1_ref)[None])
        return jnp.maximum(h, 0.0).reshape(L * B, H)

    w = wait(0); uvA = _dott(twtb(twA_ref, tbA_ref), w); start(4)
    w = wait(1); uvV = _dott(twtb(twV_ref, tbV_ref), w); start(5)
    w = wait(2); uvI = _dott(twtb(twI_ref, tbI_ref), w); start(6)

    h1A = h1_layer(sA, uvA, b1A_ref)
    h1V = h1_layer(sV, uvV, b1V_ref)
    w = wait(3); h2A = jnp.maximum(_dott(h1A, w) + _row(b2A_ref), 0.0); cp_ih.start()
    w = wait(4); h2V = jnp.maximum(_dott(h1V, w) + _row(b2V_ref), 0.0); start(7)
    w = wait(5); encA = _dott(h2A, w) + _row(b3A_ref); start(8)
    w = wait(6); encV = _dott(h2V, w) + _row(b3V_ref); start(9)
    cp_ih.wait()
    xs[...] = (_dott(encA, wih_buf[:, :H]) + _dott(encV, wih_buf[:, H:])
               + _row(bih_ref))

    h1e = jnp.maximum(sI * uvI[0:1, :] + cI * uvI[1:2, :] + _row(b1I_ref), 0.0)
    w = wait(7); h2e = jnp.maximum(_dott(h1e, w) + _row(b2I_ref), 0.0); start(10)
    w = wait(8); embI = _dott(h2e, w) + _row(b3I_ref); start(11)
    w = wait(9); state = _dott(embI, w) + _row(bsi_ref); start(12)

    whh_v = wait(10); start(13)
    cp_o.start()
    bhh = _row(bhh_ref)

    def step(t, h):
        x_t = xs[t * B:(t + 1) * B, :]
        hn = jnp.tanh(x_t + _dott(h, whh_v) + bhh)
        return jnp.where(glc > t, hn, h)

    h = state
    for t in range(0, 10):
        h = step(t, h)
    wn = wait(11); uvO = _dott(twtb(twO_ref, tbO_ref), wn)
    h1eO = jnp.maximum(sO * uvO[0:1, :] + cO * uvO[1:2, :] + _row(b1O_ref), 0.0)
    for t in range(10, 18):
        h = step(t, h)
    wn = wait(12); h2eO = jnp.maximum(_dott(h1eO, wn) + _row(b2O_ref), 0.0)
    for t in range(18, 26):
        h = step(t, h)
    wn = wait(13); embO = _dott(h2eO, wn) + _row(b3O_ref)
    for t in range(26, L):
        h = step(t, h)
    cp_o.wait()
    outpart = _dott(embO, wo_buf[:, R:])

    out_ref[...] = _dott(h, wo_buf[:, :R]) + outpart + _row(bo_ref)


@functools.partial(jax.jit, static_argnames=())
def kernel(A_data, V_data, output_data, input_data, gnn_layers, layer_parameters,
           output_lengths, input_lengths, params):
    p = params
    f32 = jnp.float32

    vspec = pl.BlockSpec(memory_space=pltpu.MemorySpace.VMEM)
    hspec = pl.BlockSpec(memory_space=pltpu.MemorySpace.HBM)

    small_ops = (
        A_data, V_data, output_data, input_data,
        gnn_layers.astype(jnp.int32), layer_parameters.astype(jnp.int32),
        output_lengths.astype(jnp.int32), input_lengths.astype(jnp.int32),
        p['A_tw'], p['A_tb'], p['V_tw'], p['V_tb'],
        p['out_tw'], p['out_tb'], p['inp_tw'], p['inp_tb'],
        p['A_b1'], p['A_b2'], p['A_b3'],
        p['V_b1'], p['V_b2'], p['V_b3'],
        p['out_b1'], p['out_b2'], p['out_b3'],
        p['inp_b1'], p['inp_b2'], p['inp_b3'],
        p['bih'], p['bsi'], p['bhh'], p['bo'],
    )
    big_ops = (
        p['A_W1'], p['V_W1'], p['out_W1'], p['inp_W1'],
        p['A_W2'], p['A_W3'], p['V_W2'], p['V_W3'],
        p['out_W2'], p['out_W3'], p['inp_W2'], p['inp_W3'],
        p['Wih'], p['Wo'], p['Wsi'], p['Whh'],
    )

    return pl.pallas_call(
        _mega_kernel,
        out_shape=jax.ShapeDtypeStruct((B, OUT), f32),
        in_specs=[vspec] * len(small_ops) + [hspec] * len(big_ops),
        out_specs=vspec,
        scratch_shapes=[
            pltpu.VMEM((NBUF, H, H), f32),
            pltpu.VMEM((R, 2 * H), f32),
            pltpu.VMEM((OUT, 2 * H), f32),
            pltpu.VMEM((L * B, R), f32),
            pltpu.SemaphoreType.DMA((NBUF,)),
            pltpu.SemaphoreType.DMA,
            pltpu.SemaphoreType.DMA,
        ],
        compiler_params=pltpu.CompilerParams(
            vmem_limit_bytes=100 * 1024 * 1024,
        ),
    )(*small_ops, *big_ops)
